# Initial kernel scaffold; baseline (speedup 1.0000x reference)
import jax
import jax.numpy as jnp
from jax.experimental import pallas as pl


def kernel(nodes, mapping, kernel, bias):
    raise NotImplementedError("write your pallas kernel here")


# speedup vs baseline: 11.2616x; 11.2616x over previous
import functools

import jax
import jax.numpy as jnp
from jax import lax
from jax.experimental import pallas as pl
from jax.experimental.pallas import tpu as pltpu
from jax.experimental.pallas import tpu_sc as plsc

_B, _V, _R, _C, _U = 4, 10000, 16, 128, 128


_VT = 1000


def _tc_transform_body(nodes_ref, w_ref, t_ref):
    x = nodes_ref[0]
    for r in range(_R):
        t_ref[0, :, r * _U:(r + 1) * _U] = jnp.dot(
            x, w_ref[r * _C:(r + 1) * _C, :],
            preferred_element_type=jnp.float32)


def _tc_transform(nodes, w):
    return pl.pallas_call(
        _tc_transform_body,
        grid=(_B, _V // _VT),
        in_specs=[
            pl.BlockSpec((1, _VT, _C), lambda b, i: (b, i, 0)),
            pl.BlockSpec((_R * _C, _U), lambda b, i: (0, 0)),
        ],
        out_specs=pl.BlockSpec((1, _VT, _R * _U), lambda b, i: (b, i, 0)),
        out_shape=jax.ShapeDtypeStruct((_B, _V, _R * _U), jnp.float32),
        compiler_params=pltpu.CompilerParams(
            dimension_semantics=("parallel", "parallel")),
    )(nodes, w)


_NW = 32
_ROWS = _B * _V
_G = 8
_NCHUNKS = _ROWS // _G
_NC_LOW = _NCHUNKS // _NW + 1


def _sc_gather_reduce_body(t_hbm, map_hbm, bias_hbm, out_hbm,
                           map_v, idx_v, rows_v, out_v, bias_v, sem):
    cid = lax.axis_index("c")
    sid = lax.axis_index("s")
    wid = sid * 2 + cid
    nrem = _NCHUNKS - (_NC_LOW - 1) * _NW
    c0 = wid * (_NC_LOW - 1) + jnp.minimum(wid, nrem)
    nc = jnp.where(wid < nrem, _NC_LOW, _NC_LOW - 1)
    pltpu.sync_copy(bias_hbm, bias_v)
    iota = lax.iota(jnp.int32, 16)

    def chunk_fn(k, carry):
        rows = pl.multiple_of((c0 + k) * _G, _G)
        bvr = (rows // _V) * (_V * _R)
        pltpu.sync_copy(map_hbm.at[pl.ds(rows * _R, _G * _R)], map_v)
        for t in range(_G):
            sl = pl.ds(t * 16, 16)
            idx_v[sl] = map_v[sl] * _R + iota + bvr
        pltpu.async_copy(t_hbm.at[idx_v], rows_v, sem).wait()

        def row_fn(g, carry2):
            base = g * 16
            for c8 in range(8):
                sl = pl.ds(c8 * 16, 16)
                acc = rows_v[base, sl]
                for j in range(1, 16):
                    acc = acc + rows_v[base + j, sl]
                out_v[g, sl] = jnp.maximum(acc + bias_v[sl], 0.0)
            return carry2

        lax.fori_loop(0, _G, row_fn, 0)
        pltpu.sync_copy(out_v, out_hbm.at[pl.ds(rows, _G)])
        return carry

    lax.fori_loop(0, nc, chunk_fn, 0)


@functools.cache
def _sc_gather_reduce():
    return pl.kernel(
        _sc_gather_reduce_body,
        out_type=jax.ShapeDtypeStruct((_ROWS, _U), jnp.float32),
        mesh=plsc.VectorSubcoreMesh(
            core_axis_name="c", subcore_axis_name="s",
            num_cores=2, num_subcores=16),
        scratch_types=[
            pltpu.VMEM((_G * _R,), jnp.int32),
            pltpu.VMEM((_G * _R,), jnp.int32),
            pltpu.VMEM((_G * _R, _U), jnp.float32),
            pltpu.VMEM((_G, _U), jnp.float32),
            pltpu.VMEM((_U,), jnp.float32),
            pltpu.SemaphoreType.DMA,
        ],
    )


def kernel(nodes, mapping, kernel, bias):
    t = _tc_transform(nodes, kernel)
    t_rows = t.reshape(_B * _V * _R, _U)
    map_flat = mapping.reshape(-1)
    out = _sc_gather_reduce()(t_rows, map_flat, bias)
    return out.reshape(_B, _V, _U)


# speedup vs baseline: 23.5495x; 2.0911x over previous
import functools

import jax
import jax.numpy as jnp
from jax import lax
from jax.experimental import pallas as pl
from jax.experimental.pallas import tpu as pltpu
from jax.experimental.pallas import tpu_sc as plsc

_B, _V, _R, _C, _U = 4, 10000, 16, 128, 128


_VT = 400


def _tc_transform_body(nodes_ref, w_ref, t_ref):
    x = nodes_ref[0]
    for r in range(_R):
        y = jnp.dot(x, w_ref[r * _C:(r + 1) * _C, :],
                    preferred_element_type=jnp.float32)
        for vb in range(_VT // 8):
            t_ref[pl.ds(vb * 128 + r * 8, 8), :] = y[vb * 8:(vb + 1) * 8, :]


def _tc_transform(nodes, w):
    return pl.pallas_call(
        _tc_transform_body,
        grid=(_B, _V // _VT),
        in_specs=[
            pl.BlockSpec((1, _VT, _C), lambda b, i: (b, i, 0)),
            pl.BlockSpec((_R * _C, _U), lambda b, i: (0, 0)),
        ],
        out_specs=pl.BlockSpec(
            (_VT * _R, _U), lambda b, i: (b * (_V // _VT) + i, 0)),
        out_shape=jax.ShapeDtypeStruct((_B * _V * _R, _U), jnp.float32),
        compiler_params=pltpu.CompilerParams(
            dimension_semantics=("parallel", "parallel")),
    )(nodes, w)


_NW = 32
_ROWS = _B * _V
_G = 8
_CPW = 160
_ROWS_PAD = _NW * _CPW * _G
_NBUF = 4
_GRP = 8


def _sc_gather_reduce_body(t_hbm, map_hbm, bias_hbm, out_hbm,
                           map_all, idx_v, rows_v, out_v, bias_v, sem):
    cid = lax.axis_index("c")
    sid = lax.axis_index("s")
    wid = sid * 2 + cid
    c0 = wid * _CPW
    pltpu.sync_copy(bias_hbm, bias_v)
    pltpu.sync_copy(map_hbm.at[pl.ds(c0 * 128, _CPW * 128)], map_all)
    iota8 = lax.iota(jnp.int32, 16) * 8

    def issue(l, b):
        rows = (c0 + l) * _G

        @pl.when(rows < _ROWS)
        def _():
            bvr = (rows // _V) * (_V * _R)
            moff = pl.multiple_of(l * 128, 128)
            for t in range(_G):
                sl = pl.ds(t * 16, 16)
                m = map_all[pl.ds(moff + t * 16, 16)]
                idx_v[b][sl] = (
                    bvr + ((m >> 3) << 7) + (m & 7) + iota8)
            pltpu.async_copy(t_hbm.at[idx_v[b]], rows_v[b], sem[b])

    def consume(l, b, orow):
        rows = (c0 + l) * _G

        @pl.when(rows < _ROWS)
        def _():
            pltpu.make_async_copy(t_hbm.at[idx_v[b]], rows_v[b], sem[b]).wait()

            def row_fn(g, carry):
                base = g * 16
                for c8 in range(8):
                    sl = pl.ds(c8 * 16, 16)
                    acc = rows_v[b][base, sl]
                    for j in range(1, 16):
                        acc = acc + rows_v[b][base + j, sl]
                    out_v[orow + g, sl] = jnp.maximum(acc + bias_v[sl], 0.0)
                return carry

            lax.fori_loop(0, _G, row_fn, 0)

    for b in range(_NBUF):
        issue(b, b)

    def group_fn(g8, carry):
        l0 = g8 * _GRP
        for cc in range(_GRP):
            b = cc % _NBUF
            consume(l0 + cc, b, cc * _G)
            issue(l0 + cc + _NBUF, b)
        grows = pl.multiple_of((c0 + l0) * _G, 64)

        @pl.when(grows < _ROWS)
        def _():
            pltpu.sync_copy(out_v, out_hbm.at[pl.ds(grows, _GRP * _G)])
        return carry

    lax.fori_loop(0, _CPW // _GRP, group_fn, 0)


@functools.cache
def _sc_gather_reduce():
    return pl.kernel(
        _sc_gather_reduce_body,
        out_type=jax.ShapeDtypeStruct((_ROWS, _U), jnp.float32),
        mesh=plsc.VectorSubcoreMesh(
            core_axis_name="c", subcore_axis_name="s",
            num_cores=2, num_subcores=16),
        compiler_params=pltpu.CompilerParams(needs_layout_passes=False),
        scratch_types=[
            pltpu.VMEM((_CPW * 128,), jnp.int32),
            [pltpu.VMEM((_G * _R,), jnp.int32)] * _NBUF,
            [pltpu.VMEM((_G * _R, _U), jnp.float32)] * _NBUF,
            pltpu.VMEM((_GRP * _G, _U), jnp.float32),
            pltpu.VMEM((_U,), jnp.float32),
            [pltpu.SemaphoreType.DMA] * _NBUF,
        ],
    )


def kernel(nodes, mapping, kernel, bias):
    t_rows = _tc_transform(nodes, kernel)
    map_flat = mapping.reshape(-1)
    map_pad = jnp.pad(map_flat, (0, (_ROWS_PAD - _ROWS) * _R))
    out = _sc_gather_reduce()(t_rows, map_pad, bias)
    return out.reshape(_B, _V, _U)


# speedup vs baseline: 27.0717x; 1.1496x over previous
import functools

import jax
import jax.numpy as jnp
from jax import lax
from jax.experimental import pallas as pl
from jax.experimental.pallas import tpu as pltpu
from jax.experimental.pallas import tpu_sc as plsc

_B, _V, _R, _C, _U = 4, 10000, 16, 128, 128


_VT = 400


def _tc_transform_body(nodes_ref, w_ref, t_ref):
    x = nodes_ref[0]
    for r in range(_R):
        y = jnp.dot(x, w_ref[r * _C:(r + 1) * _C, :],
                    preferred_element_type=jnp.float32)
        for vb in range(_VT // 8):
            t_ref[pl.ds(vb * 128 + r * 8, 8), :] = y[vb * 8:(vb + 1) * 8, :]


def _tc_transform(nodes, w, b):
    return pl.pallas_call(
        _tc_transform_body,
        grid=(_V // _VT,),
        in_specs=[
            pl.BlockSpec((1, _VT, _C), lambda i: (b, i, 0)),
            pl.BlockSpec((_R * _C, _U), lambda i: (0, 0)),
        ],
        out_specs=pl.BlockSpec((_VT * _R, _U), lambda i: (i, 0)),
        out_shape=jax.ShapeDtypeStruct((_V * _R, _U), jnp.float32),
        compiler_params=pltpu.CompilerParams(
            dimension_semantics=("parallel",)),
    )(nodes, w)


_NW = 32
_G = 8
_CPW = 40
_VPAD = _NW * _CPW * _G
_NBUF = 4
_GRP = 8


def _sc_gather_reduce_body(t_hbm, map_hbm, bias_hbm, out_hbm,
                           map_all, idx_v, rows_v, out_v, bias_v, sem):
    cid = lax.axis_index("c")
    sid = lax.axis_index("s")
    wid = sid * 2 + cid
    c0 = wid * _CPW
    pltpu.sync_copy(bias_hbm, bias_v)
    pltpu.sync_copy(map_hbm.at[pl.ds(c0 * 128, _CPW * 128)], map_all)
    iota8 = lax.iota(jnp.int32, 16) * 8

    def issue(l, b):
        rows = (c0 + l) * _G

        @pl.when(rows < _V)
        def _():
            moff = pl.multiple_of(l * 128, 128)
            for t in range(_G):
                sl = pl.ds(t * 16, 16)
                m = map_all[pl.ds(moff + t * 16, 16)]
                idx_v[b][sl] = ((m >> 3) << 7) + (m & 7) + iota8
            pltpu.async_copy(t_hbm.at[idx_v[b]], rows_v[b], sem[b])

    def consume(l, b, orow):
        rows = (c0 + l) * _G

        @pl.when(rows < _V)
        def _():
            pltpu.make_async_copy(t_hbm.at[idx_v[b]], rows_v[b], sem[b]).wait()

            def row_fn(g, carry):
                base = g * 16
                for c8 in range(8):
                    sl = pl.ds(c8 * 16, 16)
                    acc = rows_v[b][base, sl]
                    for j in range(1, 16):
                        acc = acc + rows_v[b][base + j, sl]
                    out_v[orow + g, sl] = jnp.maximum(acc + bias_v[sl], 0.0)
                return carry

            lax.fori_loop(0, _G, row_fn, 0)

    for b in range(_NBUF):
        issue(b, b)

    def group_fn(g8, carry):
        l0 = g8 * _GRP
        for cc in range(_GRP):
            b = cc % _NBUF
            consume(l0 + cc, b, cc * _G)
            issue(l0 + cc + _NBUF, b)
        grows = pl.multiple_of((c0 + l0) * _G, 64)

        @pl.when(grows + _GRP * _G <= _V)
        def _():
            pltpu.sync_copy(out_v, out_hbm.at[pl.ds(grows, _GRP * _G)])

        @pl.when(grows == (_V // 64) * 64)
        def _():
            pltpu.sync_copy(out_v.at[pl.ds(0, _V % 64)],
                            out_hbm.at[pl.ds(grows, _V % 64)])
        return carry

    lax.fori_loop(0, _CPW // _GRP, group_fn, 0)


@functools.cache
def _sc_gather_reduce():
    return pl.kernel(
        _sc_gather_reduce_body,
        out_type=jax.ShapeDtypeStruct((_V, _U), jnp.float32),
        mesh=plsc.VectorSubcoreMesh(
            core_axis_name="c", subcore_axis_name="s",
            num_cores=2, num_subcores=16),
        compiler_params=pltpu.CompilerParams(needs_layout_passes=False),
        scratch_types=[
            pltpu.VMEM((_CPW * 128,), jnp.int32),
            [pltpu.VMEM((_G * _R,), jnp.int32)] * _NBUF,
            [pltpu.VMEM((_G * _R, _U), jnp.float32)] * _NBUF,
            pltpu.VMEM((_GRP * _G, _U), jnp.float32),
            pltpu.VMEM((_U,), jnp.float32),
            [pltpu.SemaphoreType.DMA] * _NBUF,
        ],
    )


def kernel(nodes, mapping, kernel, bias):
    sc = _sc_gather_reduce()
    outs = []
    for b in range(_B):
        t_b = _tc_transform(nodes, kernel, b)
        map_b = jnp.pad(mapping[b].reshape(-1), (0, (_VPAD - _V) * _R))
        if outs:
            map_b, _ = lax.optimization_barrier((map_b, outs[-1]))
        outs.append(sc(t_b, map_b, bias))
    return jnp.stack(outs)
